# Initial kernel scaffold; baseline (speedup 1.0000x reference)
import jax
import jax.numpy as jnp
from jax.experimental import pallas as pl


def kernel(x, edge_index, sequences, g_Wih_f, g_Whh_f, g_bih_f, g_bhh_f, g_Wih_b, g_Whh_b, g_bih_b, g_bhh_b, s_Wih_f, s_Whh_f, s_bih_f, s_bhh_f, s_Wih_b, s_Whh_b, s_bih_b, s_bhh_b, mg_W0, mg_b0, mg_W1, mg_b1, mg_W2, mg_b2, ms_W0, ms_b0, ms_W1, ms_b1, ms_W2, ms_b2):
    raise NotImplementedError("write your pallas kernel here")


# speedup vs baseline: 1.0006x; 1.0006x over previous
import jax
import jax.numpy as jnp
from jax.experimental import pallas as pl

N = 10000
B = 16
NPG = N // B
ALPHA = 0.1
K = 16


def _gru_dir(xs, Wih, Whh, bih, bhh):
    H = Whh.shape[1]
    gx = jnp.einsum('btd,gd->btg', xs, Wih) + bih
    def step(h, g):
        gh = h @ Whh.T + bhh
        rx, zx, nx = jnp.split(g, 3, axis=-1)
        rh, zh, nh = jnp.split(gh, 3, axis=-1)
        r = jax.nn.sigmoid(rx + rh)
        z = jax.nn.sigmoid(zx + zh)
        n = jnp.tanh(nx + r * nh)
        h_new = (1.0 - z) * n + z * h
        return h_new, h_new
    h0 = jnp.zeros((xs.shape[0], H), xs.dtype)
    _, ys = jax.lax.scan(step, h0, jnp.swapaxes(gx, 0, 1))
    return jnp.swapaxes(ys, 0, 1)


def _bigru(xs, Wf, Uf, bif, bhf, Wb, Ub, bib, bhb):
    yf = _gru_dir(xs, Wf, Uf, bif, bhf)
    yb = _gru_dir(xs[:, ::-1, :], Wb, Ub, bib, bhb)[:, ::-1, :]
    return jnp.concatenate([yf, yb], axis=-1)


def _appnp(feat, src, dst):
    deg = jax.ops.segment_sum(jnp.ones(src.shape[0], jnp.float32), dst, num_segments=N)
    norm = jnp.where(deg > 0, deg, 1.0) ** -0.5
    h0 = feat
    h = feat
    for _ in range(K):
        h = h * norm[:, None]
        h = jax.ops.segment_sum(h[src], dst, num_segments=N)
        h = h * norm[:, None]
        h = (1.0 - ALPHA) * h + ALPHA * h0
    return h


def _mlp_kernel(v_ref, W0_ref, b0_ref, W1_ref, b1_ref, W2_ref, b2_ref, o_ref):
    v = v_ref[...]
    v = jnp.maximum(v @ W0_ref[...].T + b0_ref[...], 0.0)
    v = jnp.maximum(v @ W1_ref[...].T + b1_ref[...], 0.0)
    o_ref[...] = v @ W2_ref[...].T + b2_ref[...]


def _mlp(v, W0, b0, W1, b1, W2, b2):
    return pl.pallas_call(
        _mlp_kernel,
        out_shape=jax.ShapeDtypeStruct((v.shape[0], W2.shape[0]), v.dtype),
    )(v, W0, b0.reshape(1, -1), W1, b1.reshape(1, -1), W2, b2.reshape(1, -1))


def kernel(x, edge_index, sequences, g_Wih_f, g_Whh_f, g_bih_f, g_bhh_f, g_Wih_b, g_Whh_b, g_bih_b, g_bhh_b, s_Wih_f, s_Whh_f, s_bih_f, s_bhh_f, s_Wih_b, s_Whh_b, s_bih_b, s_bhh_b, mg_W0, mg_b0, mg_W1, mg_b1, mg_W2, mg_b2, ms_W0, ms_b0, ms_W1, ms_b1, ms_W2, ms_b2):
    seq = _bigru(sequences, s_Wih_f, s_Whh_f, s_bih_f, s_bhh_f, s_Wih_b, s_Whh_b, s_bih_b, s_bhh_b)
    seq1 = jnp.mean(seq, axis=1)
    seq2 = jnp.max(seq, axis=1)
    st = x.reshape(B, NPG, -1)
    st = _bigru(st, g_Wih_f, g_Whh_f, g_bih_f, g_bhh_f, g_Wih_b, g_Whh_b, g_bih_b, g_bhh_b)
    feat = st.reshape(N, -1)
    loops = jnp.arange(N, dtype=edge_index.dtype)
    src = jnp.concatenate([edge_index[0], loops])
    dst = jnp.concatenate([edge_index[1], loops])
    h = _appnp(feat, src, dst)
    st = h.reshape(B, NPG, -1)
    st1 = jnp.max(st, axis=1)
    st2 = jnp.mean(st, axis=1)
    graph_outputs = _mlp(st1 + st2, mg_W0, mg_b0, mg_W1, mg_b1, mg_W2, mg_b2)
    seq_outputs = _mlp(seq1 + seq2, ms_W0, ms_b0, ms_W1, ms_b1, ms_W2, ms_b2)
    return graph_outputs + seq_outputs


# speedup vs baseline: 3.0675x; 3.0657x over previous
import functools

import jax
import jax.numpy as jnp
from jax import lax
from jax.experimental import pallas as pl
from jax.experimental.pallas import tpu as pltpu
from jax.experimental.pallas import tpu_sc as plsc

N = 10000
E = 160000
B = 16
NPG = N // B
ALPHA = 0.1
K = 16

NCORE = 2
NSUB = 16
NP = 10240
STRIPE = NP // NSUB
NCHUNK = STRIPE // 128
DUMMY = 10100
EPT = 10240
EC = EPT // 128
FQ = 64
NV = FQ // 16


def _appnp_body(feat_hbm, srcT, dstT, p_hbm,
                src_v0, src_v1, dst_v, bA, bB, bF, bZ, nmat, cmat, acc, semg):
    cid = lax.axis_index("c")
    sid = lax.axis_index("s")
    rowbase = sid * STRIPE

    pltpu.sync_copy(srcT.at[sid], src_v0)
    pltpu.sync_copy(srcT.at[sid], src_v1)
    pltpu.sync_copy(dstT.at[sid], dst_v)
    off0 = ((2 * cid) * NP).astype(jnp.int32)
    off1 = ((2 * cid + 1) * NP).astype(jnp.int32)

    def _offrow(i, _):
        for u in range(8):
            sl = pl.ds(u * 16, 16)
            src_v0[i, sl] = src_v0[i, sl] + off0
            src_v1[i, sl] = src_v1[i, sl] + off1
        return 0
    lax.fori_loop(0, EC, _offrow, 0)

    zv = jnp.zeros((16,), jnp.float32)
    ov = jnp.ones((16,), jnp.float32)

    def _fill(i, _):
        for u in range(NV):
            sl = pl.ds(u * 16, 16)
            bZ[i, sl] = zv
            bA[i, sl] = ov
        return 0
    lax.fori_loop(0, 128, _fill, 0)

    def _zchunk(cki, _):
        pltpu.sync_copy(bZ, acc.at[pl.ds(rowbase + cki * 128, 128)])
        return 0
    lax.fori_loop(0, NCHUNK, _zchunk, 0)
    plsc.subcore_barrier()

    def _degchunk(j, _):
        pltpu.sync_copy(bA, acc.at[dst_v.at[j]], add=True)
        return 0
    lax.fori_loop(0, EC, _degchunk, 0)
    plsc.subcore_barrier()

    def _extchunk(cki, _):
        base = rowbase + cki * 128
        pltpu.sync_copy(acc.at[pl.ds(base, 128)], bF)

        def _extrow(r, _):
            deg = bF[r, pl.ds(0, 16)] + 1.0
            ib = lax.bitcast_convert_type(deg, jnp.int32)
            ib = 0x5F3759DF - (ib >> 1)
            y = lax.bitcast_convert_type(ib, jnp.float32)
            for _r in range(4):
                y = y * (1.5 - 0.5 * deg * y * y)
            row = cki * 128 + r
            nmat[row, pl.ds(0, 16)] = y
            cmat[row, pl.ds(0, 16)] = 0.9 / deg
            return 0
        lax.fori_loop(0, 128, _extrow, 0)
        pltpu.sync_copy(bZ, acc.at[pl.ds(base, 128)])
        return 0
    lax.fori_loop(0, NCHUNK, _extchunk, 0)

    def _pinit(cki, _):
        for q in range(2):
            gbase = (2 * cid + q) * NP + rowbase + cki * 128

            pltpu.sync_copy(feat_hbm.at[pl.ds(gbase, 128)], bF)

            def _prow(r, _):
                ns = nmat[cki * 128 + r, pl.ds(0, 16)]
                for u in range(NV):
                    sl = pl.ds(u * 16, 16)
                    bF[r, sl] = bF[r, sl] * ns
                return 0
            lax.fori_loop(0, 128, _prow, 0)
            pltpu.sync_copy(bF, p_hbm.at[pl.ds(gbase, 128)])
        return 0
    lax.fori_loop(0, NCHUNK, _pinit, 0)
    plsc.subcore_barrier()

    def _iter(k, _):
        for q in range(2):
            src_q = src_v0 if q == 0 else src_v1
            qoff = (2 * cid + q) * NP

            def _pair(jj, _):
                j0 = jj * 2
                j1 = j0 + 1
                c0 = pltpu.async_copy(p_hbm.at[src_q.at[j0]], bA, semg)
                c0.wait()
                c1 = pltpu.async_copy(p_hbm.at[src_q.at[j1]], bB, semg)
                pltpu.sync_copy(bA, acc.at[dst_v.at[j0]], add=True)
                c1.wait()
                pltpu.sync_copy(bB, acc.at[dst_v.at[j1]], add=True)
                return 0
            lax.fori_loop(0, EC // 2, _pair, 0)
            plsc.subcore_barrier()

            def _upd(cki, _):
                base = rowbase + cki * 128
                gbase = qoff + base

                pltpu.sync_copy(acc.at[pl.ds(base, 128)], bA)
                pltpu.sync_copy(p_hbm.at[pl.ds(gbase, 128)], bB)
                pltpu.sync_copy(feat_hbm.at[pl.ds(gbase, 128)], bF)

                def _urow(r, _):
                    row = cki * 128 + r
                    cs = cmat[row, pl.ds(0, 16)]
                    ns = nmat[row, pl.ds(0, 16)]
                    nb = 0.1 * ns
                    inv = jnp.where(k == K - 1, 1.0 / ns,
                                    jnp.ones((16,), jnp.float32))
                    for u in range(NV):
                        sl = pl.ds(u * 16, 16)
                        t = (bA[r, sl] + bB[r, sl]) * cs + bF[r, sl] * nb
                        bA[r, sl] = t * inv
                    return 0
                lax.fori_loop(0, 128, _urow, 0)
                pltpu.sync_copy(bA, p_hbm.at[pl.ds(gbase, 128)])
                pltpu.sync_copy(bZ, acc.at[pl.ds(base, 128)])
                return 0
            lax.fori_loop(0, NCHUNK, _upd, 0)
            plsc.subcore_barrier()
        return 0
    lax.fori_loop(0, K, _iter, 0)


def _appnp_sc(feat4, srcT, dstT):
    mesh = plsc.VectorSubcoreMesh(core_axis_name="c", subcore_axis_name="s",
                                  num_cores=NCORE, num_subcores=NSUB)
    return pl.kernel(
        _appnp_body,
        out_type=jax.ShapeDtypeStruct((4 * NP, FQ), jnp.float32),
        mesh=mesh,
        compiler_params=pltpu.CompilerParams(use_tc_tiling_on_sc=False),
        scratch_types=[
            pltpu.VMEM((EC, 128), jnp.int32),
            pltpu.VMEM((EC, 128), jnp.int32),
            pltpu.VMEM((EC, 128), jnp.int32),
            pltpu.VMEM((128, FQ), jnp.float32),
            pltpu.VMEM((128, FQ), jnp.float32),
            pltpu.VMEM((128, FQ), jnp.float32),
            pltpu.VMEM((128, FQ), jnp.float32),
            pltpu.VMEM((STRIPE, 16), jnp.float32),
            pltpu.VMEM((STRIPE, 16), jnp.float32),
            pltpu.VMEM_SHARED((NP, FQ), jnp.float32),
            pltpu.SemaphoreType.DMA,
        ],
    )(feat4, srcT, dstT)


def _gru_dir(xs, Wih, Whh, bih, bhh):
    H = Whh.shape[1]
    gx = jnp.einsum('btd,gd->btg', xs, Wih) + bih
    def step(h, g):
        gh = h @ Whh.T + bhh
        rx, zx, nx = jnp.split(g, 3, axis=-1)
        rh, zh, nh = jnp.split(gh, 3, axis=-1)
        r = jax.nn.sigmoid(rx + rh)
        z = jax.nn.sigmoid(zx + zh)
        n = jnp.tanh(nx + r * nh)
        h_new = (1.0 - z) * n + z * h
        return h_new, h_new
    h0 = jnp.zeros((xs.shape[0], H), xs.dtype)
    _, ys = jax.lax.scan(step, h0, jnp.swapaxes(gx, 0, 1))
    return jnp.swapaxes(ys, 0, 1)


def _bigru(xs, Wf, Uf, bif, bhf, Wb, Ub, bib, bhb):
    yf = _gru_dir(xs, Wf, Uf, bif, bhf)
    yb = _gru_dir(xs[:, ::-1, :], Wb, Ub, bib, bhb)[:, ::-1, :]
    return jnp.concatenate([yf, yb], axis=-1)


def _mlp(v, W0, b0, W1, b1, W2, b2):
    v = jax.nn.relu(v @ W0.T + b0)
    v = jax.nn.relu(v @ W1.T + b1)
    return v @ W2.T + b2


def kernel(x, edge_index, sequences, g_Wih_f, g_Whh_f, g_bih_f, g_bhh_f, g_Wih_b, g_Whh_b, g_bih_b, g_bhh_b, s_Wih_f, s_Whh_f, s_bih_f, s_bhh_f, s_Wih_b, s_Whh_b, s_bih_b, s_bhh_b, mg_W0, mg_b0, mg_W1, mg_b1, mg_W2, mg_b2, ms_W0, ms_b0, ms_W1, ms_b1, ms_W2, ms_b2):
    seq = _bigru(sequences, s_Wih_f, s_Whh_f, s_bih_f, s_bhh_f, s_Wih_b, s_Whh_b, s_bih_b, s_bhh_b)
    seq1 = jnp.mean(seq, axis=1)
    seq2 = jnp.max(seq, axis=1)
    st = x.reshape(B, NPG, -1)
    st = _bigru(st, g_Wih_f, g_Whh_f, g_bih_f, g_bhh_f, g_Wih_b, g_Whh_b, g_bih_b, g_bhh_b)
    feat = st.reshape(N, -1)

    feat4 = jnp.zeros((4, NP, FQ), jnp.float32)
    featq = feat.reshape(N, 4, FQ).transpose(1, 0, 2)
    feat4 = feat4.at[:, :N, :].set(featq).reshape(4 * NP, FQ)
    src = edge_index[0]
    dst = edge_index[1]
    pad = jnp.full((EPT * NSUB - E,), DUMMY, jnp.int32)
    srcT = jnp.concatenate([src, pad]).reshape(NSUB, EC, 128)
    dstT = jnp.concatenate([dst, pad]).reshape(NSUB, EC, 128)
    p = _appnp_sc(feat4, srcT, dstT)
    h = p.reshape(4, NP, FQ)[:, :N, :].transpose(1, 0, 2).reshape(N, 4 * FQ)

    st = h.reshape(B, NPG, -1)
    st1 = jnp.max(st, axis=1)
    st2 = jnp.mean(st, axis=1)
    graph_outputs = _mlp(st1 + st2, mg_W0, mg_b0, mg_W1, mg_b1, mg_W2, mg_b2)
    seq_outputs = _mlp(seq1 + seq2, ms_W0, ms_b0, ms_W1, ms_b1, ms_W2, ms_b2)
    return graph_outputs + seq_outputs


# speedup vs baseline: 4.1420x; 1.3503x over previous
import functools

import jax
import jax.numpy as jnp
from jax import lax
from jax.experimental import pallas as pl
from jax.experimental.pallas import tpu as pltpu
from jax.experimental.pallas import tpu_sc as plsc

N = 10000
E = 160000
B = 16
NPG = N // B
ALPHA = 0.1
K = 16

NCORE = 2
NSUB = 16
NP = 10240
STRIPE = NP // NSUB
NCHUNK = STRIPE // 128
DUMMY = 10100
EPT = 10240
EC = EPT // 128
FQ = 64
NV = FQ // 16


def _appnp_body(feat_hbm, srcT, dstT, p_hbm,
                src_v0, src_v1, dst_v, bA, bB, bF, bZ, nmat, cmat, acc, semg):
    cid = lax.axis_index("c")
    sid = lax.axis_index("s")
    rowbase = sid * STRIPE

    pltpu.sync_copy(srcT.at[sid], src_v0)
    pltpu.sync_copy(srcT.at[sid], src_v1)
    pltpu.sync_copy(dstT.at[sid], dst_v)
    off0 = ((2 * cid) * NP).astype(jnp.int32)
    off1 = ((2 * cid + 1) * NP).astype(jnp.int32)

    def _offrow(i, _):
        for u in range(8):
            sl = pl.ds(u * 16, 16)
            src_v0[i, sl] = src_v0[i, sl] + off0
            src_v1[i, sl] = src_v1[i, sl] + off1
        return 0
    lax.fori_loop(0, EC, _offrow, 0)

    zv = jnp.zeros((16,), jnp.float32)
    ov = jnp.ones((16,), jnp.float32)

    def _fill(i, _):
        for u in range(NV):
            sl = pl.ds(u * 16, 16)
            bZ[i, sl] = zv
            bA[i, sl] = ov
        return 0
    lax.fori_loop(0, 128, _fill, 0)

    def _zchunk(cki, _):
        pltpu.sync_copy(bZ, acc.at[pl.ds(rowbase + cki * 128, 128)])
        return 0
    lax.fori_loop(0, NCHUNK, _zchunk, 0)
    plsc.subcore_barrier()

    def _degchunk(j, _):
        pltpu.sync_copy(bA, acc.at[dst_v.at[j]], add=True)
        return 0
    lax.fori_loop(0, EC, _degchunk, 0)
    plsc.subcore_barrier()

    def _extchunk(cki, _):
        base = rowbase + cki * 128
        pltpu.sync_copy(acc.at[pl.ds(base, 128)], bF)

        def _extrow(r, _):
            deg = bF[r, pl.ds(0, 16)] + 1.0
            ib = lax.bitcast_convert_type(deg, jnp.int32)
            ib = 0x5F3759DF - (ib >> 1)
            y = lax.bitcast_convert_type(ib, jnp.float32)
            for _r in range(4):
                y = y * (1.5 - 0.5 * deg * y * y)
            row = cki * 128 + r
            nmat[row, pl.ds(0, 16)] = y
            cmat[row, pl.ds(0, 16)] = 0.9 / deg
            return 0
        lax.fori_loop(0, 128, _extrow, 0)
        pltpu.sync_copy(bZ, acc.at[pl.ds(base, 128)])
        return 0
    lax.fori_loop(0, NCHUNK, _extchunk, 0)

    def _pinit(cki, _):
        for q in range(2):
            gbase = (2 * cid + q) * NP + rowbase + cki * 128

            pltpu.sync_copy(feat_hbm.at[pl.ds(gbase, 128)], bF)

            def _prow(r, _):
                ns = nmat[cki * 128 + r, pl.ds(0, 16)]
                for u in range(NV):
                    sl = pl.ds(u * 16, 16)
                    bF[r, sl] = bF[r, sl] * ns
                return 0
            lax.fori_loop(0, 128, _prow, 0)
            pltpu.sync_copy(bF, p_hbm.at[pl.ds(gbase, 128)])
        return 0
    lax.fori_loop(0, NCHUNK, _pinit, 0)
    plsc.subcore_barrier()

    def _iter(k, _):
        for q in range(2):
            src_q = src_v0 if q == 0 else src_v1
            qoff = (2 * cid + q) * NP

            def _pair(jj, _):
                j0 = jj * 2
                j1 = j0 + 1
                c0 = pltpu.async_copy(p_hbm.at[src_q.at[j0]], bA, semg)
                c0.wait()
                c1 = pltpu.async_copy(p_hbm.at[src_q.at[j1]], bB, semg)
                pltpu.sync_copy(bA, acc.at[dst_v.at[j0]], add=True)
                c1.wait()
                pltpu.sync_copy(bB, acc.at[dst_v.at[j1]], add=True)
                return 0
            lax.fori_loop(0, EC // 2, _pair, 0)
            plsc.subcore_barrier()

            def _upd(cki, _):
                base = rowbase + cki * 128
                gbase = qoff + base

                pltpu.sync_copy(acc.at[pl.ds(base, 128)], bA)
                pltpu.sync_copy(p_hbm.at[pl.ds(gbase, 128)], bB)
                pltpu.sync_copy(feat_hbm.at[pl.ds(gbase, 128)], bF)

                def _urow(r, _):
                    row = cki * 128 + r
                    cs = cmat[row, pl.ds(0, 16)]
                    ns = nmat[row, pl.ds(0, 16)]
                    nb = 0.1 * ns
                    inv = jnp.where(k == K - 1, 1.0 / ns,
                                    jnp.ones((16,), jnp.float32))
                    for u in range(NV):
                        sl = pl.ds(u * 16, 16)
                        t = (bA[r, sl] + bB[r, sl]) * cs + bF[r, sl] * nb
                        bA[r, sl] = t * inv
                    return 0
                lax.fori_loop(0, 128, _urow, 0)
                pltpu.sync_copy(bA, p_hbm.at[pl.ds(gbase, 128)])
                pltpu.sync_copy(bZ, acc.at[pl.ds(base, 128)])
                return 0
            lax.fori_loop(0, NCHUNK, _upd, 0)
            plsc.subcore_barrier()
        return 0
    lax.fori_loop(0, K, _iter, 0)


def _appnp_sc(feat4, srcT, dstT):
    mesh = plsc.VectorSubcoreMesh(core_axis_name="c", subcore_axis_name="s",
                                  num_cores=NCORE, num_subcores=NSUB)
    return pl.kernel(
        _appnp_body,
        out_type=jax.ShapeDtypeStruct((4 * NP, FQ), jnp.float32),
        mesh=mesh,
        compiler_params=pltpu.CompilerParams(use_tc_tiling_on_sc=False),
        scratch_types=[
            pltpu.VMEM((EC, 128), jnp.int32),
            pltpu.VMEM((EC, 128), jnp.int32),
            pltpu.VMEM((EC, 128), jnp.int32),
            pltpu.VMEM((128, FQ), jnp.float32),
            pltpu.VMEM((128, FQ), jnp.float32),
            pltpu.VMEM((128, FQ), jnp.float32),
            pltpu.VMEM((128, FQ), jnp.float32),
            pltpu.VMEM((STRIPE, 16), jnp.float32),
            pltpu.VMEM((STRIPE, 16), jnp.float32),
            pltpu.VMEM_SHARED((NP, FQ), jnp.float32),
            pltpu.SemaphoreType.DMA,
        ],
    )(feat4, srcT, dstT)


def _gru_gates(gx, gh, h, H):
    r = jax.nn.sigmoid(gx[:, :H] + gh[:, :H])
    z = jax.nn.sigmoid(gx[:, H:2 * H] + gh[:, H:2 * H])
    n = jnp.tanh(gx[:, 2 * H:] + r * gh[:, 2 * H:])
    return (1.0 - z) * n + z * h


def _seq_body(L, H, xf_ref, xb_ref, wif_ref, whf_ref, bif_ref, bhf_ref,
              wib_ref, whb_ref, bib_ref, bhb_ref,
              of_ref, ob_ref, hf, hb, sf, sb, mf, mb):
    t = pl.program_id(0)

    @pl.when(t == 0)
    def _():
        hf[...] = jnp.zeros_like(hf)
        hb[...] = jnp.zeros_like(hb)

    xf = xf_ref[0]
    xb = xb_ref[0]
    gxf = xf @ wif_ref[...] + bif_ref[...]
    ghf = hf[...] @ whf_ref[...] + bhf_ref[...]
    hf_new = _gru_gates(gxf, ghf, hf[...], H)
    hf[...] = hf_new
    gxb = xb @ wib_ref[...] + bib_ref[...]
    ghb = hb[...] @ whb_ref[...] + bhb_ref[...]
    hb_new = _gru_gates(gxb, ghb, hb[...], H)
    hb[...] = hb_new

    @pl.when(t == 0)
    def _():
        sf[...] = hf_new
        sb[...] = hb_new
        mf[...] = hf_new
        mb[...] = hb_new

    @pl.when(t > 0)
    def _():
        sf[...] = sf[...] + hf_new
        sb[...] = sb[...] + hb_new
        mf[...] = jnp.maximum(mf[...], hf_new)
        mb[...] = jnp.maximum(mb[...], hb_new)

    @pl.when(t == L - 1)
    def _():
        of_ref[...] = sf[...] * (1.0 / L) + mf[...]
        ob_ref[...] = sb[...] * (1.0 / L) + mb[...]


def _seq_bigru_pooled(xs, Wf, Uf, bif, bhf, Wb, Ub, bib, bhb):
    Bb, L, D = xs.shape
    H = Uf.shape[1]
    xt = xs.transpose(1, 0, 2)
    body = functools.partial(_seq_body, L, H)
    return pl.pallas_call(
        body,
        grid=(L,),
        in_specs=[
            pl.BlockSpec((1, Bb, D), lambda t: (t, 0, 0)),
            pl.BlockSpec((1, Bb, D), lambda t: (L - 1 - t, 0, 0)),
            pl.BlockSpec((D, 3 * H), lambda t: (0, 0)),
            pl.BlockSpec((H, 3 * H), lambda t: (0, 0)),
            pl.BlockSpec((1, 3 * H), lambda t: (0, 0)),
            pl.BlockSpec((1, 3 * H), lambda t: (0, 0)),
            pl.BlockSpec((D, 3 * H), lambda t: (0, 0)),
            pl.BlockSpec((H, 3 * H), lambda t: (0, 0)),
            pl.BlockSpec((1, 3 * H), lambda t: (0, 0)),
            pl.BlockSpec((1, 3 * H), lambda t: (0, 0)),
        ],
        out_specs=[
            pl.BlockSpec((Bb, H), lambda t: (0, 0)),
            pl.BlockSpec((Bb, H), lambda t: (0, 0)),
        ],
        out_shape=[
            jax.ShapeDtypeStruct((Bb, H), jnp.float32),
            jax.ShapeDtypeStruct((Bb, H), jnp.float32),
        ],
        scratch_shapes=[pltpu.VMEM((Bb, H), jnp.float32) for _ in range(6)],
    )(xt, xt, Wf.T, Uf.T, bif.reshape(1, -1), bhf.reshape(1, -1),
      Wb.T, Ub.T, bib.reshape(1, -1), bhb.reshape(1, -1))


def _graph_body(L, H, xf_ref, xb_ref, wif_ref, whf_ref, bif_ref, bhf_ref,
                wib_ref, whb_ref, bib_ref, bhb_ref,
                yf_ref, yb_ref, hf, hb):
    t = pl.program_id(0)

    @pl.when(t == 0)
    def _():
        hf[...] = jnp.zeros_like(hf)
        hb[...] = jnp.zeros_like(hb)

    xf = xf_ref[0]
    xb = xb_ref[0]
    gxf = xf @ wif_ref[...] + bif_ref[...]
    ghf = hf[...] @ whf_ref[...] + bhf_ref[...]
    hf_new = _gru_gates(gxf, ghf, hf[...], H)
    hf[...] = hf_new
    gxb = xb @ wib_ref[...] + bib_ref[...]
    ghb = hb[...] @ whb_ref[...] + bhb_ref[...]
    hb_new = _gru_gates(gxb, ghb, hb[...], H)
    hb[...] = hb_new
    yf_ref[0] = hf_new
    yb_ref[0] = hb_new


def _graph_bigru(xs, Wf, Uf, bif, bhf, Wb, Ub, bib, bhb):
    Bb, L, D = xs.shape
    H = Uf.shape[1]
    xt = xs.transpose(1, 0, 2)
    body = functools.partial(_graph_body, L, H)
    return pl.pallas_call(
        body,
        grid=(L,),
        in_specs=[
            pl.BlockSpec((1, Bb, D), lambda t: (t, 0, 0)),
            pl.BlockSpec((1, Bb, D), lambda t: (L - 1 - t, 0, 0)),
            pl.BlockSpec((D, 3 * H), lambda t: (0, 0)),
            pl.BlockSpec((H, 3 * H), lambda t: (0, 0)),
            pl.BlockSpec((1, 3 * H), lambda t: (0, 0)),
            pl.BlockSpec((1, 3 * H), lambda t: (0, 0)),
            pl.BlockSpec((D, 3 * H), lambda t: (0, 0)),
            pl.BlockSpec((H, 3 * H), lambda t: (0, 0)),
            pl.BlockSpec((1, 3 * H), lambda t: (0, 0)),
            pl.BlockSpec((1, 3 * H), lambda t: (0, 0)),
        ],
        out_specs=[
            pl.BlockSpec((1, Bb, H), lambda t: (t, 0, 0)),
            pl.BlockSpec((1, Bb, H), lambda t: (L - 1 - t, 0, 0)),
        ],
        out_shape=[
            jax.ShapeDtypeStruct((L, Bb, H), jnp.float32),
            jax.ShapeDtypeStruct((L, Bb, H), jnp.float32),
        ],
        scratch_shapes=[pltpu.VMEM((Bb, H), jnp.float32) for _ in range(2)],
    )(xt, xt, Wf.T, Uf.T, bif.reshape(1, -1), bhf.reshape(1, -1),
      Wb.T, Ub.T, bib.reshape(1, -1), bhb.reshape(1, -1))


def _pool_body(npg, h_ref, o_ref):
    hb = h_ref[...]
    o_ref[...] = jnp.max(hb, axis=1) + jnp.sum(hb, axis=1) * (1.0 / npg)


def _graph_pool(h):
    Bb, npg, D = h.shape
    return pl.pallas_call(
        functools.partial(_pool_body, npg),
        out_shape=jax.ShapeDtypeStruct((Bb, D), jnp.float32),
    )(h)


def _mlp2_body(gv_ref, sv_ref,
               gW0, gb0, gW1, gb1, gW2, gb2,
               sW0, sb0, sW1, sb1, sW2, sb2, o_ref):
    g = jnp.maximum(gv_ref[...] @ gW0[...] + gb0[...], 0.0)
    g = jnp.maximum(g @ gW1[...] + gb1[...], 0.0)
    g = g @ gW2[...] + gb2[...]
    s = jnp.maximum(sv_ref[...] @ sW0[...] + sb0[...], 0.0)
    s = jnp.maximum(s @ sW1[...] + sb1[...], 0.0)
    s = s @ sW2[...] + sb2[...]
    o_ref[...] = g + s


def _readout(gv, sv, mg_W0, mg_b0, mg_W1, mg_b1, mg_W2, mg_b2,
             ms_W0, ms_b0, ms_W1, ms_b1, ms_W2, ms_b2):
    Bb = gv.shape[0]
    nc = mg_W2.shape[0]
    return pl.pallas_call(
        _mlp2_body,
        out_shape=jax.ShapeDtypeStruct((Bb, nc), jnp.float32),
    )(gv, sv, mg_W0.T, mg_b0.reshape(1, -1), mg_W1.T, mg_b1.reshape(1, -1),
      mg_W2.T, mg_b2.reshape(1, -1), ms_W0.T, ms_b0.reshape(1, -1),
      ms_W1.T, ms_b1.reshape(1, -1), ms_W2.T, ms_b2.reshape(1, -1))


def kernel(x, edge_index, sequences, g_Wih_f, g_Whh_f, g_bih_f, g_bhh_f, g_Wih_b, g_Whh_b, g_bih_b, g_bhh_b, s_Wih_f, s_Whh_f, s_bih_f, s_bhh_f, s_Wih_b, s_Whh_b, s_bih_b, s_bhh_b, mg_W0, mg_b0, mg_W1, mg_b1, mg_W2, mg_b2, ms_W0, ms_b0, ms_W1, ms_b1, ms_W2, ms_b2):
    ovf, ovb = _seq_bigru_pooled(sequences, s_Wih_f, s_Whh_f, s_bih_f, s_bhh_f,
                                 s_Wih_b, s_Whh_b, s_bih_b, s_bhh_b)
    seqv = jnp.concatenate([ovf, ovb], axis=1)
    yf, yb = _graph_bigru(x.reshape(B, NPG, -1), g_Wih_f, g_Whh_f, g_bih_f,
                          g_bhh_f, g_Wih_b, g_Whh_b, g_bih_b, g_bhh_b)
    feat = jnp.concatenate([yf, yb], axis=2).transpose(1, 0, 2).reshape(N, -1)

    feat4 = jnp.zeros((4, NP, FQ), jnp.float32)
    featq = feat.reshape(N, 4, FQ).transpose(1, 0, 2)
    feat4 = feat4.at[:, :N, :].set(featq).reshape(4 * NP, FQ)
    src = edge_index[0]
    dst = edge_index[1]
    pad = jnp.full((EPT * NSUB - E,), DUMMY, jnp.int32)
    srcT = jnp.concatenate([src, pad]).reshape(NSUB, EC, 128)
    dstT = jnp.concatenate([dst, pad]).reshape(NSUB, EC, 128)
    p = _appnp_sc(feat4, srcT, dstT)
    h = p.reshape(4, NP, FQ)[:, :N, :].transpose(1, 0, 2).reshape(N, 4 * FQ)

    gv = _graph_pool(h.reshape(B, NPG, -1))
    return _readout(gv, seqv, mg_W0, mg_b0, mg_W1, mg_b1, mg_W2, mg_b2,
                    ms_W0, ms_b0, ms_W1, ms_b1, ms_W2, ms_b2)


# speedup vs baseline: 5.1849x; 1.2518x over previous
import functools

import jax
import jax.numpy as jnp
from jax import lax
from jax.experimental import pallas as pl
from jax.experimental.pallas import tpu as pltpu
from jax.experimental.pallas import tpu_sc as plsc

N = 10000
E = 160000
B = 16
NPG = N // B
ALPHA = 0.1
K = 16

NCORE = 2
NSUB = 16
NP = 10240
STRIPE = NP // NSUB
NCHUNK = STRIPE // 128
DUMMY = 10100
EPT = 10240
EC = EPT // 128
FQ = 64
NV = FQ // 16


def _appnp_body(feat_hbm, srcT, dstT, p_hbm,
                src_v0, src_v1, dst_v, bA, bB, bC, bD, bF, bZ, nmat, cmat,
                acc, gs0, gs1, gs2, gs3):
    cid = lax.axis_index("c")
    sid = lax.axis_index("s")
    rowbase = sid * STRIPE

    pltpu.sync_copy(srcT.at[sid], src_v0)
    pltpu.sync_copy(srcT.at[sid], src_v1)
    pltpu.sync_copy(dstT.at[sid], dst_v)
    off0 = ((2 * cid) * NP).astype(jnp.int32)
    off1 = ((2 * cid + 1) * NP).astype(jnp.int32)

    def _offrow(i, _):
        for u in range(8):
            sl = pl.ds(u * 16, 16)
            src_v0[i, sl] = src_v0[i, sl] + off0
            src_v1[i, sl] = src_v1[i, sl] + off1
        return 0
    lax.fori_loop(0, EC, _offrow, 0)

    zv = jnp.zeros((16,), jnp.float32)
    ov = jnp.ones((16,), jnp.float32)

    def _fill(i, _):
        for u in range(NV):
            sl = pl.ds(u * 16, 16)
            bZ[i, sl] = zv
            bA[i, sl] = ov
        return 0
    lax.fori_loop(0, 128, _fill, 0)

    def _zchunk(cki, _):
        pltpu.sync_copy(bZ, acc.at[pl.ds(rowbase + cki * 128, 128)])
        return 0
    lax.fori_loop(0, NCHUNK, _zchunk, 0)
    plsc.subcore_barrier()

    def _degchunk(j, _):
        pltpu.sync_copy(bA, acc.at[dst_v.at[j]], add=True)
        return 0
    lax.fori_loop(0, EC, _degchunk, 0)
    plsc.subcore_barrier()

    def _extchunk(cki, _):
        base = rowbase + cki * 128
        pltpu.sync_copy(acc.at[pl.ds(base, 128)], bF)

        def _extrow(r, _):
            deg = bF[r, pl.ds(0, 16)] + 1.0
            ib = lax.bitcast_convert_type(deg, jnp.int32)
            ib = 0x5F3759DF - (ib >> 1)
            y = lax.bitcast_convert_type(ib, jnp.float32)
            for _r in range(4):
                y = y * (1.5 - 0.5 * deg * y * y)
            row = cki * 128 + r
            nmat[row, pl.ds(0, 16)] = y
            cmat[row, pl.ds(0, 16)] = 0.9 / deg
            return 0
        lax.fori_loop(0, 128, _extrow, 0)
        pltpu.sync_copy(bZ, acc.at[pl.ds(base, 128)])
        return 0
    lax.fori_loop(0, NCHUNK, _extchunk, 0)

    def _pinit(cki, _):
        for q in range(2):
            gbase = (2 * cid + q) * NP + rowbase + cki * 128

            pltpu.sync_copy(feat_hbm.at[pl.ds(gbase, 128)], bF)

            def _prow(r, _):
                ns = nmat[cki * 128 + r, pl.ds(0, 16)]
                for u in range(NV):
                    sl = pl.ds(u * 16, 16)
                    bF[r, sl] = bF[r, sl] * ns
                return 0
            lax.fori_loop(0, 128, _prow, 0)
            pltpu.sync_copy(bF, p_hbm.at[pl.ds(gbase, 128)])
        return 0
    lax.fori_loop(0, NCHUNK, _pinit, 0)
    plsc.subcore_barrier()

    bufs = (bA, bB, bC, bD)
    gsems = (gs0, gs1, gs2, gs3)

    def _iter(k, _):
        for q in range(2):
            src_q = src_v0 if q == 0 else src_v1
            qoff = (2 * cid + q) * NP

            pltpu.async_copy(p_hbm.at[src_q.at[0]], bufs[0], gsems[0])

            def _block(jj, _):
                for u in range(2):
                    j = jj * 2 + u

                    @pl.when(j + 1 < EC)
                    def _():
                        pltpu.async_copy(p_hbm.at[src_q.at[j + 1]],
                                         bufs[1 - u], gsems[1 - u])
                    pltpu.make_async_copy(p_hbm.at[src_q.at[j]], bufs[u],
                                          gsems[u]).wait()
                    pltpu.sync_copy(bufs[u], acc.at[dst_v.at[j]], add=True)
                return 0
            lax.fori_loop(0, EC // 2, _block, 0)
            plsc.subcore_barrier()

            def _upd(cki, _):
                base = rowbase + cki * 128
                gbase = qoff + base

                cb = pltpu.async_copy(p_hbm.at[pl.ds(gbase, 128)], bB, gs1)
                cc = pltpu.async_copy(feat_hbm.at[pl.ds(gbase, 128)], bF, gs2)
                pltpu.sync_copy(acc.at[pl.ds(base, 128)], bA)
                cb.wait()
                cc.wait()

                def _urow(r, _):
                    row = cki * 128 + r
                    cs = cmat[row, pl.ds(0, 16)]
                    ns = nmat[row, pl.ds(0, 16)]
                    nb = 0.1 * ns
                    inv = jnp.where(k == K - 1, 1.0 / ns,
                                    jnp.ones((16,), jnp.float32))
                    for u in range(NV):
                        sl = pl.ds(u * 16, 16)
                        t = (bA[r, sl] + bB[r, sl]) * cs + bF[r, sl] * nb
                        bA[r, sl] = t * inv
                    return 0
                lax.fori_loop(0, 128, _urow, 0)
                pltpu.sync_copy(bA, p_hbm.at[pl.ds(gbase, 128)])
                pltpu.sync_copy(bZ, acc.at[pl.ds(base, 128)])
                return 0
            lax.fori_loop(0, NCHUNK, _upd, 0)
            plsc.subcore_barrier()
        return 0
    lax.fori_loop(0, K, _iter, 0)


def _appnp_sc(feat4, srcT, dstT):
    mesh = plsc.VectorSubcoreMesh(core_axis_name="c", subcore_axis_name="s",
                                  num_cores=NCORE, num_subcores=NSUB)
    return pl.kernel(
        _appnp_body,
        out_type=jax.ShapeDtypeStruct((4 * NP, FQ), jnp.float32),
        mesh=mesh,
        compiler_params=pltpu.CompilerParams(use_tc_tiling_on_sc=False),
        scratch_types=[
            pltpu.VMEM((EC, 128), jnp.int32),
            pltpu.VMEM((EC, 128), jnp.int32),
            pltpu.VMEM((EC, 128), jnp.int32),
            pltpu.VMEM((128, FQ), jnp.float32),
            pltpu.VMEM((128, FQ), jnp.float32),
            pltpu.VMEM((128, FQ), jnp.float32),
            pltpu.VMEM((128, FQ), jnp.float32),
            pltpu.VMEM((128, FQ), jnp.float32),
            pltpu.VMEM((128, FQ), jnp.float32),
            pltpu.VMEM((STRIPE, 16), jnp.float32),
            pltpu.VMEM((STRIPE, 16), jnp.float32),
            pltpu.VMEM_SHARED((NP, FQ), jnp.float32),
        ] + [pltpu.SemaphoreType.DMA] * 4,
    )(feat4, srcT, dstT)


def _gru_gates(gx, gh, h, H):
    r = jax.nn.sigmoid(gx[:, :H] + gh[:, :H])
    z = jax.nn.sigmoid(gx[:, H:2 * H] + gh[:, H:2 * H])
    n = jnp.tanh(gx[:, 2 * H:] + r * gh[:, 2 * H:])
    return (1.0 - z) * n + z * h


def _seq_body(L, H, xf_ref, xb_ref, wif_ref, whf_ref, bif_ref, bhf_ref,
              wib_ref, whb_ref, bib_ref, bhb_ref,
              of_ref, ob_ref, hf, hb, sf, sb, mf, mb):
    t = pl.program_id(0)

    @pl.when(t == 0)
    def _():
        hf[...] = jnp.zeros_like(hf)
        hb[...] = jnp.zeros_like(hb)

    xf = xf_ref[0]
    xb = xb_ref[0]
    gxf = xf @ wif_ref[...] + bif_ref[...]
    ghf = hf[...] @ whf_ref[...] + bhf_ref[...]
    hf_new = _gru_gates(gxf, ghf, hf[...], H)
    hf[...] = hf_new
    gxb = xb @ wib_ref[...] + bib_ref[...]
    ghb = hb[...] @ whb_ref[...] + bhb_ref[...]
    hb_new = _gru_gates(gxb, ghb, hb[...], H)
    hb[...] = hb_new

    @pl.when(t == 0)
    def _():
        sf[...] = hf_new
        sb[...] = hb_new
        mf[...] = hf_new
        mb[...] = hb_new

    @pl.when(t > 0)
    def _():
        sf[...] = sf[...] + hf_new
        sb[...] = sb[...] + hb_new
        mf[...] = jnp.maximum(mf[...], hf_new)
        mb[...] = jnp.maximum(mb[...], hb_new)

    @pl.when(t == L - 1)
    def _():
        of_ref[...] = sf[...] * (1.0 / L) + mf[...]
        ob_ref[...] = sb[...] * (1.0 / L) + mb[...]


def _seq_bigru_pooled(xs, Wf, Uf, bif, bhf, Wb, Ub, bib, bhb):
    Bb, L, D = xs.shape
    H = Uf.shape[1]
    xt = xs.transpose(1, 0, 2)
    body = functools.partial(_seq_body, L, H)
    return pl.pallas_call(
        body,
        grid=(L,),
        in_specs=[
            pl.BlockSpec((1, Bb, D), lambda t: (t, 0, 0)),
            pl.BlockSpec((1, Bb, D), lambda t: (L - 1 - t, 0, 0)),
            pl.BlockSpec((D, 3 * H), lambda t: (0, 0)),
            pl.BlockSpec((H, 3 * H), lambda t: (0, 0)),
            pl.BlockSpec((1, 3 * H), lambda t: (0, 0)),
            pl.BlockSpec((1, 3 * H), lambda t: (0, 0)),
            pl.BlockSpec((D, 3 * H), lambda t: (0, 0)),
            pl.BlockSpec((H, 3 * H), lambda t: (0, 0)),
            pl.BlockSpec((1, 3 * H), lambda t: (0, 0)),
            pl.BlockSpec((1, 3 * H), lambda t: (0, 0)),
        ],
        out_specs=[
            pl.BlockSpec((Bb, H), lambda t: (0, 0)),
            pl.BlockSpec((Bb, H), lambda t: (0, 0)),
        ],
        out_shape=[
            jax.ShapeDtypeStruct((Bb, H), jnp.float32),
            jax.ShapeDtypeStruct((Bb, H), jnp.float32),
        ],
        scratch_shapes=[pltpu.VMEM((Bb, H), jnp.float32) for _ in range(6)],
    )(xt, xt, Wf.T, Uf.T, bif.reshape(1, -1), bhf.reshape(1, -1),
      Wb.T, Ub.T, bib.reshape(1, -1), bhb.reshape(1, -1))


def _graph_body(L, H, xf_ref, xb_ref, wif_ref, whf_ref, bif_ref, bhf_ref,
                wib_ref, whb_ref, bib_ref, bhb_ref,
                yf_ref, yb_ref, hf, hb):
    t = pl.program_id(0)

    @pl.when(t == 0)
    def _():
        hf[...] = jnp.zeros_like(hf)
        hb[...] = jnp.zeros_like(hb)

    xf = xf_ref[0]
    xb = xb_ref[0]
    gxf = xf @ wif_ref[...] + bif_ref[...]
    ghf = hf[...] @ whf_ref[...] + bhf_ref[...]
    hf_new = _gru_gates(gxf, ghf, hf[...], H)
    hf[...] = hf_new
    gxb = xb @ wib_ref[...] + bib_ref[...]
    ghb = hb[...] @ whb_ref[...] + bhb_ref[...]
    hb_new = _gru_gates(gxb, ghb, hb[...], H)
    hb[...] = hb_new
    yf_ref[0] = hf_new
    yb_ref[0] = hb_new


def _graph_bigru(xs, Wf, Uf, bif, bhf, Wb, Ub, bib, bhb):
    Bb, L, D = xs.shape
    H = Uf.shape[1]
    xt = xs.transpose(1, 0, 2)
    body = functools.partial(_graph_body, L, H)
    return pl.pallas_call(
        body,
        grid=(L,),
        in_specs=[
            pl.BlockSpec((1, Bb, D), lambda t: (t, 0, 0)),
            pl.BlockSpec((1, Bb, D), lambda t: (L - 1 - t, 0, 0)),
            pl.BlockSpec((D, 3 * H), lambda t: (0, 0)),
            pl.BlockSpec((H, 3 * H), lambda t: (0, 0)),
            pl.BlockSpec((1, 3 * H), lambda t: (0, 0)),
            pl.BlockSpec((1, 3 * H), lambda t: (0, 0)),
            pl.BlockSpec((D, 3 * H), lambda t: (0, 0)),
            pl.BlockSpec((H, 3 * H), lambda t: (0, 0)),
            pl.BlockSpec((1, 3 * H), lambda t: (0, 0)),
            pl.BlockSpec((1, 3 * H), lambda t: (0, 0)),
        ],
        out_specs=[
            pl.BlockSpec((1, Bb, H), lambda t: (t, 0, 0)),
            pl.BlockSpec((1, Bb, H), lambda t: (L - 1 - t, 0, 0)),
        ],
        out_shape=[
            jax.ShapeDtypeStruct((L, Bb, H), jnp.float32),
            jax.ShapeDtypeStruct((L, Bb, H), jnp.float32),
        ],
        scratch_shapes=[pltpu.VMEM((Bb, H), jnp.float32) for _ in range(2)],
    )(xt, xt, Wf.T, Uf.T, bif.reshape(1, -1), bhf.reshape(1, -1),
      Wb.T, Ub.T, bib.reshape(1, -1), bhb.reshape(1, -1))


def _pool_body(npg, h_ref, o_ref):
    hb = h_ref[...]
    o_ref[...] = jnp.max(hb, axis=1) + jnp.sum(hb, axis=1) * (1.0 / npg)


def _graph_pool(h):
    Bb, npg, D = h.shape
    return pl.pallas_call(
        functools.partial(_pool_body, npg),
        out_shape=jax.ShapeDtypeStruct((Bb, D), jnp.float32),
    )(h)


def _mlp2_body(gv_ref, sv_ref,
               gW0, gb0, gW1, gb1, gW2, gb2,
               sW0, sb0, sW1, sb1, sW2, sb2, o_ref):
    g = jnp.maximum(gv_ref[...] @ gW0[...] + gb0[...], 0.0)
    g = jnp.maximum(g @ gW1[...] + gb1[...], 0.0)
    g = g @ gW2[...] + gb2[...]
    s = jnp.maximum(sv_ref[...] @ sW0[...] + sb0[...], 0.0)
    s = jnp.maximum(s @ sW1[...] + sb1[...], 0.0)
    s = s @ sW2[...] + sb2[...]
    o_ref[...] = g + s


def _readout(gv, sv, mg_W0, mg_b0, mg_W1, mg_b1, mg_W2, mg_b2,
             ms_W0, ms_b0, ms_W1, ms_b1, ms_W2, ms_b2):
    Bb = gv.shape[0]
    nc = mg_W2.shape[0]
    return pl.pallas_call(
        _mlp2_body,
        out_shape=jax.ShapeDtypeStruct((Bb, nc), jnp.float32),
    )(gv, sv, mg_W0.T, mg_b0.reshape(1, -1), mg_W1.T, mg_b1.reshape(1, -1),
      mg_W2.T, mg_b2.reshape(1, -1), ms_W0.T, ms_b0.reshape(1, -1),
      ms_W1.T, ms_b1.reshape(1, -1), ms_W2.T, ms_b2.reshape(1, -1))


def kernel(x, edge_index, sequences, g_Wih_f, g_Whh_f, g_bih_f, g_bhh_f, g_Wih_b, g_Whh_b, g_bih_b, g_bhh_b, s_Wih_f, s_Whh_f, s_bih_f, s_bhh_f, s_Wih_b, s_Whh_b, s_bih_b, s_bhh_b, mg_W0, mg_b0, mg_W1, mg_b1, mg_W2, mg_b2, ms_W0, ms_b0, ms_W1, ms_b1, ms_W2, ms_b2):
    ovf, ovb = _seq_bigru_pooled(sequences, s_Wih_f, s_Whh_f, s_bih_f, s_bhh_f,
                                 s_Wih_b, s_Whh_b, s_bih_b, s_bhh_b)
    seqv = jnp.concatenate([ovf, ovb], axis=1)
    yf, yb = _graph_bigru(x.reshape(B, NPG, -1), g_Wih_f, g_Whh_f, g_bih_f,
                          g_bhh_f, g_Wih_b, g_Whh_b, g_bih_b, g_bhh_b)
    feat = jnp.concatenate([yf, yb], axis=2).transpose(1, 0, 2).reshape(N, -1)

    feat4 = jnp.zeros((4, NP, FQ), jnp.float32)
    featq = feat.reshape(N, 4, FQ).transpose(1, 0, 2)
    feat4 = feat4.at[:, :N, :].set(featq).reshape(4 * NP, FQ)
    src = edge_index[0]
    dst = edge_index[1]
    pad = jnp.full((EPT * NSUB - E,), DUMMY, jnp.int32)
    srcT = jnp.concatenate([src, pad]).reshape(NSUB, EC, 128)
    dstT = jnp.concatenate([dst, pad]).reshape(NSUB, EC, 128)
    p = _appnp_sc(feat4, srcT, dstT)
    h = p.reshape(4, NP, FQ)[:, :N, :].transpose(1, 0, 2).reshape(N, 4 * FQ)

    gv = _graph_pool(h.reshape(B, NPG, -1))
    return _readout(gv, seqv, mg_W0, mg_b0, mg_W1, mg_b1, mg_W2, mg_b2,
                    ms_W0, ms_b0, ms_W1, ms_b1, ms_W2, ms_b2)
